# Initial kernel scaffold; baseline (speedup 1.0000x reference)
import jax
import jax.numpy as jnp
from jax.experimental import pallas as pl


def kernel(x, lookup_table_weight, height, width):
    raise NotImplementedError("write your pallas kernel here")


# speedup vs baseline: 28.7151x; 28.7151x over previous
import functools
import math

import numpy as np
import jax
import jax.numpy as jnp
from jax import lax
from jax.experimental import pallas as pl
from jax.experimental.pallas import tpu as pltpu
from jax.experimental.pallas import tpu_sc as plsc

ALPHA, BETA, GAMMA = 1.9, 3.8, 15.2
B, H, L, D = 8, 12, 576, 64
K = 49
KP = 64
BH = B * H

NC, NS = 2, 16
NW = NC * NS
BH_G = 4
I_G = NW // BH_G
BH_PER = BH // BH_G
I_PER = L // I_G
CH = 24
I_CHUNKS = I_PER // CH
LANES = 16
JV = L // LANES


def _piecewise_index_np(rp, alpha, beta, gamma):
    rp = np.asarray(rp, dtype=np.float64)
    rp_abs = np.abs(rp)
    mask = rp_abs <= alpha
    safe = np.maximum(rp_abs, 1e-12)
    y = np.sign(rp) * np.minimum(
        np.round(alpha + np.log(safe / alpha) / math.log(gamma / alpha) * (beta - alpha)),
        beta,
    )
    idx = np.round(rp).astype(np.int64)
    idx[~mask] = y[~mask].astype(np.int64)
    return idx


def _bucket_ids_product(height, width, alpha, beta, gamma):
    rows = np.arange(height).reshape(height, 1).repeat(width, axis=1)
    cols = np.arange(width).reshape(1, width).repeat(height, axis=0)
    pos = np.stack([rows, cols], axis=2).reshape(-1, 2)
    diff = pos[:, None, :] - pos[None, :, :]
    beta_int = int(beta)
    S = 2 * beta_int + 1
    r = _piecewise_index_np(diff[:, :, 0], alpha, beta, gamma) + beta_int
    c = _piecewise_index_np(diff[:, :, 1], alpha, beta, gamma) + beta_int
    return (r * S + c).astype(np.int32)


_BUCKET_BASE = _bucket_ids_product(24, 24, ALPHA, BETA, GAMMA)


def _mm_body(x_ref, w_ref, o_ref):
    xm = x_ref[:, 0].reshape(B * L, D)
    o = jnp.dot(xm, w_ref[0], preferred_element_type=jnp.float32)
    o_ref[:, 0] = o.reshape(B, L, KP)


def _tc_einsum(x, w_pad):
    return pl.pallas_call(
        _mm_body,
        grid=(H,),
        in_specs=[
            pl.BlockSpec((B, 1, L, D), lambda h: (0, h, 0, 0)),
            pl.BlockSpec((1, D, KP), lambda h: (h, 0, 0)),
        ],
        out_specs=pl.BlockSpec((B, 1, L, KP), lambda h: (0, h, 0, 0)),
        out_shape=jax.ShapeDtypeStruct((B, H, L, KP), jnp.float32),
    )(x, w_pad)


def _sc_body(lt_ref, bid_ref, out_ref, idx_v, lt_v, out_v):
    c = lax.axis_index("c")
    s = lax.axis_index("s")
    wid = s * NC + c
    bh0 = (wid % BH_G) * BH_PER
    i0 = (wid // BH_G) * I_PER

    def chunk_body(ci, carry):
        ibase = i0 + ci * CH
        pltpu.sync_copy(bid_ref.at[pl.ds(ibase, CH)], idx_v)

        def bh_body(bh, carry2):
            b = bh0 + bh
            pltpu.sync_copy(lt_ref.at[b, pl.ds(ibase, CH)], lt_v)

            def i_body(il, carry3):
                row = jnp.full((LANES,), il, dtype=jnp.int32)
                for j in range(JV):
                    cidx = idx_v[il, pl.ds(j * LANES, LANES)]
                    vals = plsc.load_gather(lt_v, [row, cidx])
                    out_v[il, pl.ds(j * LANES, LANES)] = vals
                return carry3

            lax.fori_loop(0, CH, i_body, 0)
            pltpu.sync_copy(out_v, out_ref.at[b, pl.ds(ibase, CH)])
            return carry2

        lax.fori_loop(0, BH_PER, bh_body, 0)
        return carry

    lax.fori_loop(0, I_CHUNKS, chunk_body, 0)


_sc_gather = functools.partial(
    pl.kernel,
    out_type=jax.ShapeDtypeStruct((BH, L, L), jnp.float32),
    mesh=plsc.VectorSubcoreMesh(core_axis_name="c", subcore_axis_name="s"),
    compiler_params=pltpu.CompilerParams(needs_layout_passes=False),
    scratch_types=[
        pltpu.VMEM((CH, L), jnp.int32),
        pltpu.VMEM((CH, KP), jnp.float32),
        pltpu.VMEM((CH, L), jnp.float32),
    ],
)(_sc_body)


def kernel(x, lookup_table_weight, height, width):
    w_pad = jnp.pad(lookup_table_weight, ((0, 0), (0, 0), (0, KP - K)))
    lt = _tc_einsum(x, w_pad)
    offset = (height - 24) + (width - 24)
    bid = jnp.asarray(_BUCKET_BASE) + jnp.asarray(offset, jnp.int32)
    out = _sc_gather(lt.reshape(BH, L, KP), bid)
    return out.reshape(B, H, L, L)


# speedup vs baseline: 67.2257x; 2.3411x over previous
import functools
import math

import numpy as np
import jax
import jax.numpy as jnp
from jax import lax
from jax.experimental import pallas as pl
from jax.experimental.pallas import tpu as pltpu
from jax.experimental.pallas import tpu_sc as plsc

ALPHA, BETA, GAMMA = 1.9, 3.8, 15.2
B, H, L, D = 8, 12, 576, 64
K = 49
KP = 64
BH = B * H

NC, NS = 2, 16
NW = NC * NS
BH_G = 4
I_G = NW // BH_G
BH_PER = BH // BH_G
I_PER = L // I_G
CH = 24
I_CHUNKS = I_PER // CH
LANES = 16
JV = L // LANES
PAIRS = BH_PER // 2


def _piecewise_index_np(rp, alpha, beta, gamma):
    rp = np.asarray(rp, dtype=np.float64)
    rp_abs = np.abs(rp)
    mask = rp_abs <= alpha
    safe = np.maximum(rp_abs, 1e-12)
    y = np.sign(rp) * np.minimum(
        np.round(alpha + np.log(safe / alpha) / math.log(gamma / alpha) * (beta - alpha)),
        beta,
    )
    idx = np.round(rp).astype(np.int64)
    idx[~mask] = y[~mask].astype(np.int64)
    return idx


def _bucket_ids_product(height, width, alpha, beta, gamma):
    rows = np.arange(height).reshape(height, 1).repeat(width, axis=1)
    cols = np.arange(width).reshape(1, width).repeat(height, axis=0)
    pos = np.stack([rows, cols], axis=2).reshape(-1, 2)
    diff = pos[:, None, :] - pos[None, :, :]
    beta_int = int(beta)
    S = 2 * beta_int + 1
    r = _piecewise_index_np(diff[:, :, 0], alpha, beta, gamma) + beta_int
    c = _piecewise_index_np(diff[:, :, 1], alpha, beta, gamma) + beta_int
    return (r * S + c).astype(np.int32)


_BUCKET_BASE = _bucket_ids_product(24, 24, ALPHA, BETA, GAMMA)
_BID2 = (_BUCKET_BASE + (np.arange(L) % CH)[:, None].astype(np.int32) * KP).reshape(-1)


LH = L // 2


def _mm_body(x_ref, w_ref, o_ref):
    xm = x_ref[:, 0].reshape(B * LH, D)
    o = jnp.dot(xm, w_ref[0], preferred_element_type=jnp.float32)
    o_ref[:, 0] = o.reshape(B, LH, KP)


def _tc_einsum(x, w_pad):
    return pl.pallas_call(
        _mm_body,
        grid=(H, 2),
        in_specs=[
            pl.BlockSpec((B, 1, LH, D), lambda h, l: (0, h, l, 0)),
            pl.BlockSpec((1, D, KP), lambda h, l: (h, 0, 0)),
        ],
        out_specs=pl.BlockSpec((B, 1, LH, KP), lambda h, l: (0, h, l, 0)),
        out_shape=jax.ShapeDtypeStruct((B, H, L, KP), jnp.float32),
    )(x, w_pad)


def _sc_body(lt_ref, bid_ref, out_ref,
             idx_v, lt0_v, lt1_v, out0_v, out1_v,
             sem_l0, sem_l1, sem_o0, sem_o1):
    c = lax.axis_index("c")
    s = lax.axis_index("s")
    wid = s * NC + c
    bh0 = (wid % BH_G) * BH_PER
    i0 = (wid // BH_G) * I_PER

    def out_dst(b, ibase):
        return out_ref.at[b // H, b % H, pl.ds(ibase, CH)]

    def gather_rows(lt_v, out_v):
        @plsc.parallel_loop(0, CH, unroll=1)
        def _(il):
            ibase_v = il * L

            @plsc.parallel_loop(0, JV, unroll=9)
            def _(jw):
                fidx = idx_v[pl.ds(ibase_v + jw * LANES, LANES)]
                out_v[il, pl.ds(jw * LANES, LANES)] = plsc.load_gather(
                    lt_v, [fidx]
                )

    def chunk_body(ci, carry):
        ibase = i0 + ci * CH
        pltpu.sync_copy(bid_ref.at[pl.ds(ibase * L, CH * L)], idx_v)
        pltpu.async_copy(
            lt_ref.at[bh0, pl.ds(ibase * KP, CH * KP)], lt0_v, sem_l0
        )

        def pair_body(t, carry2):
            b0 = bh0 + 2 * t
            b1 = b0 + 1

            def half(b_cur, b_next, lt_cur, lt_next, sem_cur, sem_next,
                     out_v, sem_o, do_prefetch):
                pltpu.make_async_copy(
                    lt_ref.at[b_cur, pl.ds(ibase * KP, CH * KP)],
                    lt_cur, sem_cur,
                ).wait()

                @pl.when(do_prefetch)
                def _():
                    pltpu.async_copy(
                        lt_ref.at[b_next, pl.ds(ibase * KP, CH * KP)],
                        lt_next, sem_next,
                    )

                gather_rows(lt_cur, out_v)
                pltpu.async_copy(
                    out_v, out_dst(b_cur, ibase), sem_o
                )

            @pl.when(t > 0)
            def _():
                pltpu.make_async_copy(
                    out0_v, out_dst(b0, ibase), sem_o0
                ).wait()
            half(b0, b1, lt0_v, lt1_v, sem_l0, sem_l1, out0_v, sem_o0,
                 do_prefetch=True)

            @pl.when(t > 0)
            def _():
                pltpu.make_async_copy(
                    out1_v, out_dst(b1, ibase), sem_o1
                ).wait()
            half(b1, bh0 + 2 * t + 2, lt1_v, lt0_v, sem_l1, sem_l0,
                 out1_v, sem_o1, do_prefetch=t < PAIRS - 1)
            return carry2

        lax.fori_loop(0, PAIRS, pair_body, 0)
        pltpu.make_async_copy(
            out0_v, out_dst(bh0, ibase), sem_o0
        ).wait()
        pltpu.make_async_copy(
            out1_v, out_dst(bh0, ibase), sem_o1
        ).wait()
        return carry

    lax.fori_loop(0, I_CHUNKS, chunk_body, 0)


_sc_gather = functools.partial(
    pl.kernel,
    out_type=jax.ShapeDtypeStruct((B, H, L, L), jnp.float32),
    mesh=plsc.VectorSubcoreMesh(core_axis_name="c", subcore_axis_name="s"),
    compiler_params=pltpu.CompilerParams(needs_layout_passes=False),
    scratch_types=[
        pltpu.VMEM((CH * L,), jnp.int32),
        pltpu.VMEM((CH * KP,), jnp.float32),
        pltpu.VMEM((CH * KP,), jnp.float32),
        pltpu.VMEM((CH, L), jnp.float32),
        pltpu.VMEM((CH, L), jnp.float32),
        pltpu.SemaphoreType.DMA,
        pltpu.SemaphoreType.DMA,
        pltpu.SemaphoreType.DMA,
        pltpu.SemaphoreType.DMA,
    ],
)(_sc_body)


def kernel(x, lookup_table_weight, height, width):
    w_pad = jnp.pad(lookup_table_weight, ((0, 0), (0, 0), (0, KP - K)))
    lt = _tc_einsum(x, w_pad)
    offset = (height - 24) + (width - 24)
    bid = jnp.asarray(_BID2) + jnp.asarray(offset, jnp.int32)
    return _sc_gather(lt.reshape(BH, L * KP), bid)


# speedup vs baseline: 70.2256x; 1.0446x over previous
import functools
import math

import numpy as np
import jax
import jax.numpy as jnp
from jax import lax
from jax.experimental import pallas as pl
from jax.experimental.pallas import tpu as pltpu
from jax.experimental.pallas import tpu_sc as plsc

ALPHA, BETA, GAMMA = 1.9, 3.8, 15.2
B, H, L, D = 8, 12, 576, 64
K = 49
KP = 64
BH = B * H

NC, NS = 2, 16
NW = NC * NS
BH_G = 4
I_G = NW // BH_G
BH_PER = BH // BH_G
I_PER = L // I_G
CH = 24
I_CHUNKS = I_PER // CH
LANES = 16
JV = L // LANES
PAIRS = BH_PER // 2


def _piecewise_index_np(rp, alpha, beta, gamma):
    rp = np.asarray(rp, dtype=np.float64)
    rp_abs = np.abs(rp)
    mask = rp_abs <= alpha
    safe = np.maximum(rp_abs, 1e-12)
    y = np.sign(rp) * np.minimum(
        np.round(alpha + np.log(safe / alpha) / math.log(gamma / alpha) * (beta - alpha)),
        beta,
    )
    idx = np.round(rp).astype(np.int64)
    idx[~mask] = y[~mask].astype(np.int64)
    return idx


def _bucket_ids_product(height, width, alpha, beta, gamma):
    rows = np.arange(height).reshape(height, 1).repeat(width, axis=1)
    cols = np.arange(width).reshape(1, width).repeat(height, axis=0)
    pos = np.stack([rows, cols], axis=2).reshape(-1, 2)
    diff = pos[:, None, :] - pos[None, :, :]
    beta_int = int(beta)
    S = 2 * beta_int + 1
    r = _piecewise_index_np(diff[:, :, 0], alpha, beta, gamma) + beta_int
    c = _piecewise_index_np(diff[:, :, 1], alpha, beta, gamma) + beta_int
    return (r * S + c).astype(np.int32)


_BUCKET_BASE = _bucket_ids_product(24, 24, ALPHA, BETA, GAMMA)
_BID2 = (_BUCKET_BASE + (np.arange(L) % CH)[:, None].astype(np.int32) * KP).reshape(-1)


def _mm_body(x_ref, w_ref, o_ref):
    xm = x_ref[:, 0].reshape(B * L, D)
    o = jnp.dot(xm, w_ref[0], preferred_element_type=jnp.float32)
    o_ref[:, 0] = o.reshape(B, L, KP)


def _tc_einsum(x, w_pad):
    return pl.pallas_call(
        _mm_body,
        grid=(H,),
        in_specs=[
            pl.BlockSpec((B, 1, L, D), lambda h: (0, h, 0, 0)),
            pl.BlockSpec((1, D, KP), lambda h: (h, 0, 0)),
        ],
        out_specs=pl.BlockSpec((B, 1, L, KP), lambda h: (0, h, 0, 0)),
        out_shape=jax.ShapeDtypeStruct((B, H, L, KP), jnp.float32),
    )(x, w_pad)


def _sc_body(lt_ref, bid_ref, out_ref,
             idx_v, lt0_v, lt1_v, out0_v, out1_v,
             sem_l0, sem_l1, sem_o0, sem_o1):
    c = lax.axis_index("c")
    s = lax.axis_index("s")
    wid = s * NC + c
    bh0 = (wid % BH_G) * BH_PER
    i0 = (wid // BH_G) * I_PER

    def out_dst(b, ibase):
        return out_ref.at[b // H, b % H, pl.ds(ibase, CH)]

    def gather_rows(lt_v, out_v):
        @plsc.parallel_loop(0, CH, unroll=1)
        def _(il):
            ibase_v = il * L

            @plsc.parallel_loop(0, JV, unroll=9)
            def _(jw):
                fidx = idx_v[pl.ds(ibase_v + jw * LANES, LANES)]
                out_v[il, pl.ds(jw * LANES, LANES)] = plsc.load_gather(
                    lt_v, [fidx]
                )

    def chunk_body(ci, carry):
        ibase = i0 + ci * CH
        pltpu.sync_copy(bid_ref.at[pl.ds(ibase * L, CH * L)], idx_v)
        pltpu.async_copy(
            lt_ref.at[bh0, pl.ds(ibase * KP, CH * KP)], lt0_v, sem_l0
        )

        def pair_body(t, carry2):
            b0 = bh0 + 2 * t
            b1 = b0 + 1
            not_first = jnp.logical_or(ci > 0, t > 0)

            def half(b_cur, b_next, lt_cur, lt_next, sem_cur, sem_next,
                     out_v, sem_o, do_prefetch):
                pltpu.make_async_copy(
                    lt_ref.at[b_cur, pl.ds(ibase * KP, CH * KP)],
                    lt_cur, sem_cur,
                ).wait()

                @pl.when(do_prefetch)
                def _():
                    pltpu.async_copy(
                        lt_ref.at[b_next, pl.ds(ibase * KP, CH * KP)],
                        lt_next, sem_next,
                    )

                gather_rows(lt_cur, out_v)
                pltpu.async_copy(
                    out_v, out_dst(b_cur, ibase), sem_o
                )

            @pl.when(not_first)
            def _():
                pltpu.make_async_copy(
                    out0_v, out_dst(b0, ibase), sem_o0
                ).wait()
            half(b0, b1, lt0_v, lt1_v, sem_l0, sem_l1, out0_v, sem_o0,
                 do_prefetch=True)

            @pl.when(not_first)
            def _():
                pltpu.make_async_copy(
                    out1_v, out_dst(b1, ibase), sem_o1
                ).wait()
            half(b1, bh0 + 2 * t + 2, lt1_v, lt0_v, sem_l1, sem_l0,
                 out1_v, sem_o1, do_prefetch=t < PAIRS - 1)
            return carry2

        lax.fori_loop(0, PAIRS, pair_body, 0)
        return carry

    lax.fori_loop(0, I_CHUNKS, chunk_body, 0)
    pltpu.make_async_copy(out0_v, out_dst(bh0, i0), sem_o0).wait()
    pltpu.make_async_copy(out1_v, out_dst(bh0, i0), sem_o1).wait()


_sc_gather = functools.partial(
    pl.kernel,
    out_type=jax.ShapeDtypeStruct((B, H, L, L), jnp.float32),
    mesh=plsc.VectorSubcoreMesh(core_axis_name="c", subcore_axis_name="s"),
    compiler_params=pltpu.CompilerParams(needs_layout_passes=False),
    scratch_types=[
        pltpu.VMEM((CH * L,), jnp.int32),
        pltpu.VMEM((CH * KP,), jnp.float32),
        pltpu.VMEM((CH * KP,), jnp.float32),
        pltpu.VMEM((CH, L), jnp.float32),
        pltpu.VMEM((CH, L), jnp.float32),
        pltpu.SemaphoreType.DMA,
        pltpu.SemaphoreType.DMA,
        pltpu.SemaphoreType.DMA,
        pltpu.SemaphoreType.DMA,
    ],
)(_sc_body)


def kernel(x, lookup_table_weight, height, width):
    w_pad = jnp.pad(lookup_table_weight, ((0, 0), (0, 0), (0, KP - K)))
    lt = _tc_einsum(x, w_pad)
    offset = (height - 24) + (width - 24)
    bid = jnp.asarray(_BID2) + jnp.asarray(offset, jnp.int32)
    return _sc_gather(lt.reshape(BH, L * KP), bid)
